# Initial kernel scaffold; baseline (speedup 1.0000x reference)
import jax
import jax.numpy as jnp
from jax.experimental import pallas as pl


def kernel(feature_emb, indexes, graph_dict, W1, b1, W2, b2, W3, b3):
    raise NotImplementedError("write your pallas kernel here")


# speedup vs baseline: 3.3927x; 3.3927x over previous
import functools

import jax
import jax.numpy as jnp
from jax import lax
from jax.experimental import pallas as pl
from jax.experimental.pallas import tpu as pltpu
from jax.experimental.pallas import tpu_sc as plsc

NUM_FIELDS = 26
FE_DIM = 512
F2 = NUM_FIELDS * NUM_FIELDS
VOCAB = 100000
B = 4096

NUM_WORKERS = 32
PAD_VOCAB = 100352
CHUNK = PAD_VOCAB // NUM_WORKERS
LANES = 16


def _mlp_kernel(x_ref, wc1_ref, c1_ref, d2_ref, c2_ref, d3_ref, c3_ref, out_ref):
    h1 = jnp.dot(x_ref[...], wc1_ref[...], preferred_element_type=jnp.float32)
    h1 = jnp.maximum(h1 + c1_ref[...], 0.0)
    h2 = jnp.dot(h1, d2_ref[...], preferred_element_type=jnp.float32)
    h2 = jnp.maximum(h2 + c2_ref[...], 0.0)
    m2 = jnp.sum(h2, axis=0, keepdims=True) * (1.0 / x_ref.shape[0])
    out_ref[...] = (
        jnp.dot(m2, d3_ref[...], preferred_element_type=jnp.float32) + c3_ref[...]
    )


def _select_kernel(flags_ref, gd_ref, gvec_ref, out_ref):
    mask = flags_ref[...] > 0
    out_ref[...] = jnp.where(mask, gvec_ref[...], gd_ref[...])


def _flags_pallas(indexes):
    mesh = plsc.VectorSubcoreMesh(core_axis_name="c", subcore_axis_name="s")

    @functools.partial(
        pl.kernel,
        mesh=mesh,
        out_type=jax.ShapeDtypeStruct((PAD_VOCAB,), jnp.int32),
        scratch_types=[
            pltpu.VMEM((B,), jnp.int32),
            pltpu.VMEM((CHUNK,), jnp.int32),
        ],
        compiler_params=pltpu.CompilerParams(needs_layout_passes=False),
    )
    def flags_kernel(idx_hbm, out_hbm, idx_v, flag_v):
        wid = lax.axis_index("s") * 2 + lax.axis_index("c")
        base = wid * CHUNK
        pltpu.sync_copy(idx_hbm, idx_v)

        def zero_body(i, carry):
            flag_v[pl.ds(i * LANES, LANES)] = jnp.zeros((LANES,), jnp.int32)
            return carry

        lax.fori_loop(0, CHUNK // LANES, zero_body, 0)

        ones = jnp.ones((LANES,), jnp.int32)

        def scat_body(j, carry):
            idxv = idx_v[pl.ds(j * LANES, LANES)]
            local = idxv - base
            m = (local >= 0) & (local < CHUNK)
            safe = jnp.clip(local, 0, CHUNK - 1)
            plsc.store_scatter(flag_v, [safe], ones, mask=m)
            return carry

        lax.fori_loop(0, B // LANES, scat_body, 0)
        pltpu.sync_copy(flag_v, out_hbm.at[pl.ds(base, CHUNK)])

    return flags_kernel(indexes)


def kernel(feature_emb, indexes, graph_dict, W1, b1, W2, b2, W3, b3):
    f = NUM_FIELDS
    Wc1 = W1[:, :FE_DIM, :].transpose(1, 0, 2).reshape(FE_DIM, F2)
    diag = W1[jnp.arange(f), FE_DIM + jnp.arange(f), :]
    c1 = (diag + b1).reshape(1, F2)
    eye = jnp.eye(f, dtype=W2.dtype)
    D2 = (W2[:, :, None, :] * eye[:, None, :, None]).reshape(F2, F2)
    D3 = (W3[:, :, None, :] * eye[:, None, :, None]).reshape(F2, F2)
    c2 = b2.reshape(1, F2)
    c3 = b3.reshape(1, F2)

    gvec = pl.pallas_call(
        _mlp_kernel,
        out_shape=jax.ShapeDtypeStruct((1, F2), jnp.float32),
    )(feature_emb, Wc1, c1, D2, c2, D3, c3)

    flags = _flags_pallas(indexes.astype(jnp.int32))
    flags2d = flags[:VOCAB].reshape(VOCAB, 1)

    rows = 2000
    new_mem = pl.pallas_call(
        _select_kernel,
        grid=(VOCAB // rows,),
        in_specs=[
            pl.BlockSpec((rows, 1), lambda i: (i, 0)),
            pl.BlockSpec((rows, F2), lambda i: (i, 0)),
            pl.BlockSpec((1, F2), lambda i: (0, 0)),
        ],
        out_specs=pl.BlockSpec((rows, F2), lambda i: (i, 0)),
        out_shape=jax.ShapeDtypeStruct((VOCAB, F2), jnp.float32),
    )(flags2d, graph_dict, gvec)
    return new_mem


# speedup vs baseline: 3.3957x; 1.0009x over previous
import functools

import jax
import jax.numpy as jnp
from jax import lax
from jax.experimental import pallas as pl
from jax.experimental.pallas import tpu as pltpu
from jax.experimental.pallas import tpu_sc as plsc

NUM_FIELDS = 26
FE_DIM = 512
F2 = NUM_FIELDS * NUM_FIELDS
VOCAB = 100000
B = 4096

NUM_WORKERS = 32
PAD_VOCAB = 100352
CHUNK = PAD_VOCAB // NUM_WORKERS
LANES = 16


def _mlp_kernel(x_ref, wc1_ref, c1_ref, d2_ref, c2_ref, d3_ref, c3_ref, out_ref):
    h1 = jnp.dot(x_ref[...], wc1_ref[...], preferred_element_type=jnp.float32)
    h1 = jnp.maximum(h1 + c1_ref[...], 0.0)
    h2 = jnp.dot(h1, d2_ref[...], preferred_element_type=jnp.float32)
    h2 = jnp.maximum(h2 + c2_ref[...], 0.0)
    m2 = jnp.sum(h2, axis=0, keepdims=True) * (1.0 / x_ref.shape[0])
    out_ref[...] = (
        jnp.dot(m2, d3_ref[...], preferred_element_type=jnp.float32) + c3_ref[...]
    )


def _select_kernel(flags_ref, gd_ref, gvec_ref, out_ref):
    mask = flags_ref[...] > 0
    out_ref[...] = jnp.where(mask, gvec_ref[...], gd_ref[...])


def _flags_pallas(indexes):
    mesh = plsc.VectorSubcoreMesh(core_axis_name="c", subcore_axis_name="s")

    @functools.partial(
        pl.kernel,
        mesh=mesh,
        out_type=jax.ShapeDtypeStruct((PAD_VOCAB,), jnp.int32),
        scratch_types=[
            pltpu.VMEM((B,), jnp.int32),
            pltpu.VMEM((CHUNK,), jnp.int32),
        ],
        compiler_params=pltpu.CompilerParams(needs_layout_passes=False),
    )
    def flags_kernel(idx_hbm, out_hbm, idx_v, flag_v):
        wid = lax.axis_index("s") * 2 + lax.axis_index("c")
        base = wid * CHUNK
        pltpu.sync_copy(idx_hbm, idx_v)

        def zero_body(i, carry):
            flag_v[pl.ds(i * LANES, LANES)] = jnp.zeros((LANES,), jnp.int32)
            return carry

        lax.fori_loop(0, CHUNK // LANES, zero_body, 0)

        ones = jnp.ones((LANES,), jnp.int32)

        def scat_body(j, carry):
            idxv = idx_v[pl.ds(j * LANES, LANES)]
            local = idxv - base
            m = (local >= 0) & (local < CHUNK)
            safe = jnp.clip(local, 0, CHUNK - 1)
            plsc.store_scatter(flag_v, [safe], ones, mask=m)
            return carry

        lax.fori_loop(0, B // LANES, scat_body, 0)
        pltpu.sync_copy(flag_v, out_hbm.at[pl.ds(base, CHUNK)])

    return flags_kernel(indexes)


def kernel(feature_emb, indexes, graph_dict, W1, b1, W2, b2, W3, b3):
    f = NUM_FIELDS
    Wc1 = W1[:, :FE_DIM, :].transpose(1, 0, 2).reshape(FE_DIM, F2)
    diag = W1[jnp.arange(f), FE_DIM + jnp.arange(f), :]
    c1 = (diag + b1).reshape(1, F2)
    eye = jnp.eye(f, dtype=W2.dtype)
    D2 = (W2[:, :, None, :] * eye[:, None, :, None]).reshape(F2, F2)
    D3 = (W3[:, :, None, :] * eye[:, None, :, None]).reshape(F2, F2)
    c2 = b2.reshape(1, F2)
    c3 = b3.reshape(1, F2)

    gvec = pl.pallas_call(
        _mlp_kernel,
        out_shape=jax.ShapeDtypeStruct((1, F2), jnp.float32),
    )(feature_emb, Wc1, c1, D2, c2, D3, c3)

    flags = _flags_pallas(indexes.astype(jnp.int32))
    flags2d = flags[:VOCAB].reshape(VOCAB, 1)

    rows = 4000
    new_mem = pl.pallas_call(
        _select_kernel,
        grid=(VOCAB // rows,),
        in_specs=[
            pl.BlockSpec((rows, 1), lambda i: (i, 0)),
            pl.BlockSpec((rows, F2), lambda i: (i, 0)),
            pl.BlockSpec((1, F2), lambda i: (0, 0)),
        ],
        out_specs=pl.BlockSpec((rows, F2), lambda i: (i, 0)),
        out_shape=jax.ShapeDtypeStruct((VOCAB, F2), jnp.float32),
    )(flags2d, graph_dict, gvec)
    return new_mem
